# baseline (device time: 49230 ns/iter reference)
import jax
import jax.numpy as jnp
from jax import lax
from jax.experimental import pallas as pl
from jax.experimental.pallas import tpu as pltpu

N_DEV = 4
B_PER = 2
SQ = 256
SKV = 256
HQ = 16
HQ_PER = 4
DH = 64
D_MODEL = 512
D_HEADS = HQ_PER * DH
WINDOW = 128
SCALE = 0.125
M2 = B_PER * SQ


def kernel(x, Wq, K_ext, V_ext, Wo):
    def body(x_ref, wq_ref, k_any, v_any, wo_ref, out_ref,
             wq_buf, wo_buf, k_scr, v_scr, wq_snd, wo_snd,
             send_sems, recv_sems, kv_sems):
        my_pos = lax.axis_index("i")
        b0 = my_pos * B_PER

        k_dmas, v_dmas = [], []
        for j in range(N_DEV):
            g = lax.rem(my_pos + j, N_DEV)
            for h in range(HQ_PER):
                head = g * HQ_PER + h
                s = j * HQ_PER + h
                ck = pltpu.make_async_copy(
                    k_any.at[pl.ds(b0, B_PER), :, head, :],
                    k_scr.at[s], kv_sems.at[0, s])
                ck.start()
                k_dmas.append(ck)
                cv = pltpu.make_async_copy(
                    v_any.at[pl.ds(b0, B_PER), :, head, :],
                    v_scr.at[s], kv_sems.at[1, s])
                cv.start()
                v_dmas.append(cv)

        wq_snd[...] = wq_ref[...].astype(jnp.bfloat16)
        wo_snd[...] = wo_ref[...].astype(jnp.bfloat16)

        barrier = pltpu.get_barrier_semaphore()
        for k in range(1, N_DEV):
            pl.semaphore_signal(
                barrier, inc=1,
                device_id=((my_pos + k) % N_DEV,),
                device_id_type=pl.DeviceIdType.MESH,
            )
        pl.semaphore_wait(barrier, N_DEV - 1)

        for i, k in enumerate([2, 1, 3]):
            peer = (my_pos + k) % N_DEV
            slot = (N_DEV - k) % N_DEV
            rq = pltpu.make_async_remote_copy(
                src_ref=wq_snd,
                dst_ref=wq_buf.at[slot],
                send_sem=send_sems.at[0, i],
                recv_sem=recv_sems.at[0, slot],
                device_id=(peer,),
                device_id_type=pl.DeviceIdType.MESH,
            )
            rq.start()
            ro = pltpu.make_async_remote_copy(
                src_ref=wo_snd,
                dst_ref=wo_buf.at[slot],
                send_sem=send_sems.at[1, i],
                recv_sem=recv_sems.at[1, slot],
                device_id=(peer,),
                device_id_type=pl.DeviceIdType.MESH,
            )
            ro.start()

        qi = lax.broadcasted_iota(jnp.int32, (SQ, SKV), 0)
        ki = lax.broadcasted_iota(jnp.int32, (SQ, SKV), 1)
        mask = jnp.abs(qi - ki) <= WINDOW

        def wait_recv(buf, t, j):
            r = pltpu.make_async_remote_copy(
                src_ref=buf.at[0],
                dst_ref=buf.at[j],
                send_sem=send_sems.at[t, 0],
                recv_sem=recv_sems.at[t, j],
                device_id=(my_pos,),
                device_id_type=pl.DeviceIdType.MESH,
            )
            r.wait_recv()

        x2 = x_ref[...].reshape(M2, D_MODEL).astype(jnp.bfloat16)
        acc = jnp.zeros((M2, D_MODEL), jnp.float32)
        for j in [0, 1, 3, 2]:
            if j == 0:
                wq_g = wq_snd[...]
            else:
                wait_recv(wq_buf, 0, j)
                wq_g = wq_buf[j]

            q_all = jnp.dot(x2, wq_g,
                            preferred_element_type=jnp.float32)
            q_all = q_all.astype(jnp.bfloat16)
            for h in range(HQ_PER):
                k_dmas[j * HQ_PER + h].wait()
                v_dmas[j * HQ_PER + h].wait()
            ctx_parts = []
            for b in range(B_PER):
                for h in range(HQ_PER):
                    s = j * HQ_PER + h
                    q = q_all[b * SQ:(b + 1) * SQ, h * DH:(h + 1) * DH]
                    kk = k_scr[s, b].astype(jnp.bfloat16)
                    sc = lax.dot_general(
                        q, kk, (((1,), (1,)), ((), ())),
                        preferred_element_type=jnp.float32,
                    ) * SCALE
                    e = jnp.exp(jnp.where(mask, sc, -1e9))
                    w = e / jnp.sum(e, axis=1, keepdims=True)
                    ctx_parts.append(jnp.dot(
                        w.astype(jnp.bfloat16),
                        v_scr[s, b].astype(jnp.bfloat16),
                        preferred_element_type=jnp.float32))
            ctx = jnp.concatenate(
                [jnp.concatenate(ctx_parts[b * HQ_PER:(b + 1) * HQ_PER], axis=1)
                 for b in range(B_PER)], axis=0)

            if j == 0:
                wo_g = wo_snd[...]
            else:
                wait_recv(wo_buf, 1, j)
                wo_g = wo_buf[j]
            acc = acc + jnp.dot(ctx.astype(jnp.bfloat16), wo_g,
                                preferred_element_type=jnp.float32)

        out_ref[...] = acc.reshape(B_PER, SQ, D_MODEL)

        for i in range(N_DEV - 1):
            dq = pltpu.make_async_remote_copy(
                src_ref=wq_snd, dst_ref=wq_buf.at[0],
                send_sem=send_sems.at[0, i],
                recv_sem=recv_sems.at[0, 0],
                device_id=(my_pos,), device_id_type=pl.DeviceIdType.MESH,
            )
            dq.wait_send()
            do = pltpu.make_async_remote_copy(
                src_ref=wo_snd, dst_ref=wo_buf.at[0],
                send_sem=send_sems.at[1, i],
                recv_sem=recv_sems.at[1, 0],
                device_id=(my_pos,), device_id_type=pl.DeviceIdType.MESH,
            )
            do.wait_send()

    return pl.pallas_call(
        body,
        out_shape=jax.ShapeDtypeStruct((B_PER, SQ, D_MODEL), jnp.float32),
        in_specs=[
            pl.BlockSpec(memory_space=pltpu.VMEM),
            pl.BlockSpec(memory_space=pltpu.VMEM),
            pl.BlockSpec(memory_space=pl.ANY),
            pl.BlockSpec(memory_space=pl.ANY),
            pl.BlockSpec(memory_space=pltpu.VMEM),
        ],
        out_specs=pl.BlockSpec(memory_space=pltpu.VMEM),
        scratch_shapes=[
            pltpu.VMEM((N_DEV, D_MODEL, D_HEADS), jnp.bfloat16),
            pltpu.VMEM((N_DEV, D_HEADS, D_MODEL), jnp.bfloat16),
            pltpu.VMEM((HQ, B_PER, SKV, DH), jnp.float32),
            pltpu.VMEM((HQ, B_PER, SKV, DH), jnp.float32),
            pltpu.VMEM((D_MODEL, D_HEADS), jnp.bfloat16),
            pltpu.VMEM((D_HEADS, D_MODEL), jnp.bfloat16),
            pltpu.SemaphoreType.DMA((2, N_DEV - 1)),
            pltpu.SemaphoreType.DMA((2, N_DEV)),
            pltpu.SemaphoreType.DMA((2, HQ)),
        ],
        compiler_params=pltpu.CompilerParams(collective_id=0),
    )(x, Wq, K_ext, V_ext, Wo)


# device time: 36127 ns/iter; 1.3627x vs baseline; 1.3627x over previous
import jax
import jax.numpy as jnp
from jax import lax
from jax.experimental import pallas as pl
from jax.experimental.pallas import tpu as pltpu

N_DEV = 4
B_PER = 2
SQ = 256
SKV = 256
HQ = 16
HQ_PER = 4
DH = 64
D_MODEL = 512
D_HEADS = HQ_PER * DH
WINDOW = 128
SCALE = 0.125
M2 = B_PER * SQ


def kernel(x, Wq, K_ext, V_ext, Wo):
    k2 = K_ext.reshape(8, SKV, HQ * DH)
    v2 = V_ext.reshape(8, SKV, HQ * DH)

    def body(x_ref, wq_ref, k_any, v_any, wo_ref, out_ref,
             wq_buf, wo_buf, k_grp, v_grp, wq_snd, wo_snd,
             send_sems, recv_sems, kv_sems):
        my_pos = lax.axis_index("i")
        b0 = my_pos * B_PER

        k_dmas, v_dmas = [], []
        for j in range(N_DEV):
            g = lax.rem(my_pos + j, N_DEV)
            ck = pltpu.make_async_copy(
                k_any.at[pl.ds(b0, B_PER), :, pl.ds(g * D_HEADS, D_HEADS)],
                k_grp.at[j], kv_sems.at[0, j])
            ck.start()
            k_dmas.append(ck)
            cv = pltpu.make_async_copy(
                v_any.at[pl.ds(b0, B_PER), :, pl.ds(g * D_HEADS, D_HEADS)],
                v_grp.at[j], kv_sems.at[1, j])
            cv.start()
            v_dmas.append(cv)

        wq_snd[...] = wq_ref[...].astype(jnp.bfloat16)
        wo_snd[...] = wo_ref[...].astype(jnp.bfloat16)

        barrier = pltpu.get_barrier_semaphore()
        for k in range(1, N_DEV):
            pl.semaphore_signal(
                barrier, inc=1,
                device_id=((my_pos + k) % N_DEV,),
                device_id_type=pl.DeviceIdType.MESH,
            )
        pl.semaphore_wait(barrier, N_DEV - 1)

        for i, k in enumerate([2, 1, 3]):
            peer = (my_pos + k) % N_DEV
            slot = (N_DEV - k) % N_DEV
            rq = pltpu.make_async_remote_copy(
                src_ref=wq_snd,
                dst_ref=wq_buf.at[slot],
                send_sem=send_sems.at[0, i],
                recv_sem=recv_sems.at[0, slot],
                device_id=(peer,),
                device_id_type=pl.DeviceIdType.MESH,
            )
            rq.start()
            ro = pltpu.make_async_remote_copy(
                src_ref=wo_snd,
                dst_ref=wo_buf.at[slot],
                send_sem=send_sems.at[1, i],
                recv_sem=recv_sems.at[1, slot],
                device_id=(peer,),
                device_id_type=pl.DeviceIdType.MESH,
            )
            ro.start()

        qi = lax.broadcasted_iota(jnp.int32, (SQ, SKV), 0)
        ki = lax.broadcasted_iota(jnp.int32, (SQ, SKV), 1)
        mask = jnp.abs(qi - ki) <= WINDOW

        def wait_recv(buf, t, j):
            r = pltpu.make_async_remote_copy(
                src_ref=buf.at[0],
                dst_ref=buf.at[j],
                send_sem=send_sems.at[t, 0],
                recv_sem=recv_sems.at[t, j],
                device_id=(my_pos,),
                device_id_type=pl.DeviceIdType.MESH,
            )
            r.wait_recv()

        x2 = x_ref[...].reshape(M2, D_MODEL).astype(jnp.bfloat16)
        acc = jnp.zeros((M2, D_MODEL), jnp.float32)
        for j in [0, 1, 3, 2]:
            if j == 0:
                wq_g = wq_snd[...]
            else:
                wait_recv(wq_buf, 0, j)
                wq_g = wq_buf[j]

            q_all = jnp.dot(x2, wq_g,
                            preferred_element_type=jnp.float32)
            q_all = q_all.astype(jnp.bfloat16)
            k_dmas[j].wait()
            v_dmas[j].wait()
            kb = [k_grp[j, b].astype(jnp.bfloat16) for b in range(B_PER)]
            vb = [v_grp[j, b].astype(jnp.bfloat16) for b in range(B_PER)]
            ctx_parts = []
            for b in range(B_PER):
                for h in range(HQ_PER):
                    q = q_all[b * SQ:(b + 1) * SQ, h * DH:(h + 1) * DH]
                    kk = kb[b][:, h * DH:(h + 1) * DH]
                    sc = lax.dot_general(
                        q, kk, (((1,), (1,)), ((), ())),
                        preferred_element_type=jnp.float32,
                    ) * SCALE
                    e = jnp.exp(jnp.where(mask, sc, -1e9))
                    w = e / jnp.sum(e, axis=1, keepdims=True)
                    ctx_parts.append(jnp.dot(
                        w.astype(jnp.bfloat16),
                        vb[b][:, h * DH:(h + 1) * DH],
                        preferred_element_type=jnp.float32))
            ctx = jnp.concatenate(
                [jnp.concatenate(ctx_parts[b * HQ_PER:(b + 1) * HQ_PER], axis=1)
                 for b in range(B_PER)], axis=0)

            if j == 0:
                wo_g = wo_snd[...]
            else:
                wait_recv(wo_buf, 1, j)
                wo_g = wo_buf[j]
            acc = acc + jnp.dot(ctx.astype(jnp.bfloat16), wo_g,
                                preferred_element_type=jnp.float32)

        out_ref[...] = acc.reshape(B_PER, SQ, D_MODEL)

        for i in range(N_DEV - 1):
            dq = pltpu.make_async_remote_copy(
                src_ref=wq_snd, dst_ref=wq_buf.at[0],
                send_sem=send_sems.at[0, i],
                recv_sem=recv_sems.at[0, 0],
                device_id=(my_pos,), device_id_type=pl.DeviceIdType.MESH,
            )
            dq.wait_send()
            do = pltpu.make_async_remote_copy(
                src_ref=wo_snd, dst_ref=wo_buf.at[0],
                send_sem=send_sems.at[1, i],
                recv_sem=recv_sems.at[1, 0],
                device_id=(my_pos,), device_id_type=pl.DeviceIdType.MESH,
            )
            do.wait_send()

    return pl.pallas_call(
        body,
        out_shape=jax.ShapeDtypeStruct((B_PER, SQ, D_MODEL), jnp.float32),
        in_specs=[
            pl.BlockSpec(memory_space=pltpu.VMEM),
            pl.BlockSpec(memory_space=pltpu.VMEM),
            pl.BlockSpec(memory_space=pl.ANY),
            pl.BlockSpec(memory_space=pl.ANY),
            pl.BlockSpec(memory_space=pltpu.VMEM),
        ],
        out_specs=pl.BlockSpec(memory_space=pltpu.VMEM),
        scratch_shapes=[
            pltpu.VMEM((N_DEV, D_MODEL, D_HEADS), jnp.bfloat16),
            pltpu.VMEM((N_DEV, D_HEADS, D_MODEL), jnp.bfloat16),
            pltpu.VMEM((N_DEV, B_PER, SKV, D_HEADS), jnp.float32),
            pltpu.VMEM((N_DEV, B_PER, SKV, D_HEADS), jnp.float32),
            pltpu.VMEM((D_MODEL, D_HEADS), jnp.bfloat16),
            pltpu.VMEM((D_HEADS, D_MODEL), jnp.bfloat16),
            pltpu.SemaphoreType.DMA((2, N_DEV - 1)),
            pltpu.SemaphoreType.DMA((2, N_DEV)),
            pltpu.SemaphoreType.DMA((2, N_DEV)),
        ],
        compiler_params=pltpu.CompilerParams(collective_id=0),
    )(x, Wq, k2, v2, Wo)


# device time: 26361 ns/iter; 1.8675x vs baseline; 1.3705x over previous
import jax
import jax.numpy as jnp
from jax import lax
from jax.experimental import pallas as pl
from jax.experimental.pallas import tpu as pltpu

N_DEV = 4
B_PER = 2
SQ = 256
SKV = 256
HQ_PER = 4
DH = 64
D_MODEL = 512
D_HEADS = HQ_PER * DH
WINDOW = 128
SCALE = 0.125
M2 = B_PER * SQ


def kernel(x, Wq, K_ext, V_ext, Wo):
    my = lax.axis_index("i")
    k_loc = lax.dynamic_slice_in_dim(K_ext, my * B_PER, B_PER, axis=0)
    v_loc = lax.dynamic_slice_in_dim(V_ext, my * B_PER, B_PER, axis=0)
    k_loc = jnp.transpose(k_loc.astype(jnp.bfloat16), (2, 0, 1, 3))
    v_loc = jnp.transpose(v_loc.astype(jnp.bfloat16), (2, 0, 1, 3))
    x_b = x.astype(jnp.bfloat16)
    wq_b = Wq.astype(jnp.bfloat16)
    wo_b = Wo.astype(jnp.bfloat16)

    def body(x_ref, wq_ref, k_ref, v_ref, wo_ref, out_ref,
             wq_buf, wo_buf, send_sems, recv_sems):
        my_pos = lax.axis_index("i")

        barrier = pltpu.get_barrier_semaphore()
        for k in range(1, N_DEV):
            pl.semaphore_signal(
                barrier, inc=1,
                device_id=((my_pos + k) % N_DEV,),
                device_id_type=pl.DeviceIdType.MESH,
            )
        pl.semaphore_wait(barrier, N_DEV - 1)

        for i, k in enumerate([1, 3, 2]):
            peer = (my_pos + k) % N_DEV
            slot = (N_DEV - k) % N_DEV
            rq = pltpu.make_async_remote_copy(
                src_ref=wq_ref,
                dst_ref=wq_buf.at[slot],
                send_sem=send_sems.at[0, i],
                recv_sem=recv_sems.at[0, slot],
                device_id=(peer,),
                device_id_type=pl.DeviceIdType.MESH,
            )
            rq.start()
            ro = pltpu.make_async_remote_copy(
                src_ref=wo_ref,
                dst_ref=wo_buf.at[slot],
                send_sem=send_sems.at[1, i],
                recv_sem=recv_sems.at[1, slot],
                device_id=(peer,),
                device_id_type=pl.DeviceIdType.MESH,
            )
            ro.start()

        qi = lax.broadcasted_iota(jnp.int32, (SQ, SKV), 0)
        ki = lax.broadcasted_iota(jnp.int32, (SQ, SKV), 1)
        mask = jnp.abs(qi - ki) <= WINDOW

        def wait_recv(buf, t, j):
            r = pltpu.make_async_remote_copy(
                src_ref=buf.at[0],
                dst_ref=buf.at[j],
                send_sem=send_sems.at[t, 0],
                recv_sem=recv_sems.at[t, j],
                device_id=(my_pos,),
                device_id_type=pl.DeviceIdType.MESH,
            )
            r.wait_recv()

        x2 = x_ref[...].reshape(M2, D_MODEL)
        acc = jnp.zeros((M2, D_MODEL), jnp.float32)
        for j in [0, 1, 3, 2]:
            if j == 0:
                wq_g = wq_ref[...]
            else:
                wait_recv(wq_buf, 0, j)
                wq_g = wq_buf[j]

            g = lax.rem(my_pos + j, N_DEV)
            q_all = jnp.dot(x2, wq_g,
                            preferred_element_type=jnp.float32)
            q_all = q_all.astype(jnp.bfloat16)
            ctx_parts = []
            for b in range(B_PER):
                for h in range(HQ_PER):
                    head = g * HQ_PER + h
                    q = q_all[b * SQ:(b + 1) * SQ, h * DH:(h + 1) * DH]
                    kk = k_ref[head, b]
                    s = lax.dot_general(
                        q, kk, (((1,), (1,)), ((), ())),
                        preferred_element_type=jnp.float32,
                    ) * SCALE
                    e = jnp.exp(jnp.where(mask, s, -1e9))
                    r = 1.0 / jnp.sum(e, axis=1, keepdims=True)
                    ctx_h = jnp.dot(e.astype(jnp.bfloat16), v_ref[head, b],
                                    preferred_element_type=jnp.float32)
                    ctx_parts.append(ctx_h * r)
            ctx = jnp.concatenate(
                [jnp.concatenate(ctx_parts[b * HQ_PER:(b + 1) * HQ_PER], axis=1)
                 for b in range(B_PER)], axis=0)

            if j == 0:
                wo_g = wo_ref[...]
            else:
                wait_recv(wo_buf, 1, j)
                wo_g = wo_buf[j]
            acc = acc + jnp.dot(ctx.astype(jnp.bfloat16), wo_g,
                                preferred_element_type=jnp.float32)

        out_ref[...] = acc.reshape(B_PER, SQ, D_MODEL)

        for i in range(N_DEV - 1):
            dq = pltpu.make_async_remote_copy(
                src_ref=wq_ref, dst_ref=wq_buf.at[0],
                send_sem=send_sems.at[0, i],
                recv_sem=recv_sems.at[0, 0],
                device_id=(my_pos,), device_id_type=pl.DeviceIdType.MESH,
            )
            dq.wait_send()
            do = pltpu.make_async_remote_copy(
                src_ref=wo_ref, dst_ref=wo_buf.at[0],
                send_sem=send_sems.at[1, i],
                recv_sem=recv_sems.at[1, 0],
                device_id=(my_pos,), device_id_type=pl.DeviceIdType.MESH,
            )
            do.wait_send()

    return pl.pallas_call(
        body,
        out_shape=jax.ShapeDtypeStruct((B_PER, SQ, D_MODEL), jnp.float32),
        in_specs=[
            pl.BlockSpec(memory_space=pltpu.VMEM),
            pl.BlockSpec(memory_space=pltpu.VMEM),
            pl.BlockSpec(memory_space=pltpu.VMEM),
            pl.BlockSpec(memory_space=pltpu.VMEM),
            pl.BlockSpec(memory_space=pltpu.VMEM),
        ],
        out_specs=pl.BlockSpec(memory_space=pltpu.VMEM),
        scratch_shapes=[
            pltpu.VMEM((N_DEV, D_MODEL, D_HEADS), jnp.bfloat16),
            pltpu.VMEM((N_DEV, D_HEADS, D_MODEL), jnp.bfloat16),
            pltpu.SemaphoreType.DMA((2, N_DEV - 1)),
            pltpu.SemaphoreType.DMA((2, N_DEV)),
        ],
        compiler_params=pltpu.CompilerParams(collective_id=0),
    )(x_b, wq_b, k_loc, v_loc, wo_b)


# device time: 24814 ns/iter; 1.9840x vs baseline; 1.0623x over previous
import jax
import jax.numpy as jnp
from jax import lax
from jax.experimental import pallas as pl
from jax.experimental.pallas import tpu as pltpu

N_DEV = 4
B_PER = 2
SQ = 256
SKV = 256
HQ_PER = 4
DH = 64
D_MODEL = 512
D_HEADS = HQ_PER * DH
WINDOW = 128
SCALE = 0.125
M2 = B_PER * SQ


def kernel(x, Wq, K_ext, V_ext, Wo):
    my = lax.axis_index("i")
    k_loc = lax.dynamic_slice_in_dim(K_ext, my * B_PER, B_PER, axis=0)
    v_loc = lax.dynamic_slice_in_dim(V_ext, my * B_PER, B_PER, axis=0)
    k_loc = jnp.transpose(k_loc.astype(jnp.bfloat16), (2, 0, 1, 3))
    v_loc = jnp.transpose(v_loc.astype(jnp.bfloat16), (2, 0, 1, 3))
    wq_b = Wq.astype(jnp.bfloat16)
    wo_b = Wo.astype(jnp.bfloat16)

    def body(x_ref, wq_ref, k_ref, v_ref, wo_ref, out_ref,
             wq_buf, wo_buf, send_sems, recv_sems):
        my_pos = lax.axis_index("i")

        barrier = pltpu.get_barrier_semaphore()
        for k in range(1, N_DEV):
            pl.semaphore_signal(
                barrier, inc=1,
                device_id=((my_pos + k) % N_DEV,),
                device_id_type=pl.DeviceIdType.MESH,
            )
        pl.semaphore_wait(barrier, N_DEV - 1)

        for i, k in enumerate([1, 3, 2]):
            peer = (my_pos + k) % N_DEV
            slot = (N_DEV - k) % N_DEV
            rq = pltpu.make_async_remote_copy(
                src_ref=wq_ref,
                dst_ref=wq_buf.at[slot],
                send_sem=send_sems.at[0, i],
                recv_sem=recv_sems.at[0, slot],
                device_id=(peer,),
                device_id_type=pl.DeviceIdType.MESH,
            )
            rq.start()
            ro = pltpu.make_async_remote_copy(
                src_ref=wo_ref,
                dst_ref=wo_buf.at[slot],
                send_sem=send_sems.at[1, i],
                recv_sem=recv_sems.at[1, slot],
                device_id=(peer,),
                device_id_type=pl.DeviceIdType.MESH,
            )
            ro.start()

        qi = lax.broadcasted_iota(jnp.int32, (SQ, SKV), 0)
        ki = lax.broadcasted_iota(jnp.int32, (SQ, SKV), 1)
        mask = jnp.abs(qi - ki) <= WINDOW

        def wait_recv(buf, t, j):
            r = pltpu.make_async_remote_copy(
                src_ref=buf.at[0],
                dst_ref=buf.at[j],
                send_sem=send_sems.at[t, 0],
                recv_sem=recv_sems.at[t, j],
                device_id=(my_pos,),
                device_id_type=pl.DeviceIdType.MESH,
            )
            r.wait_recv()

        x2 = x_ref[...].reshape(M2, D_MODEL).astype(jnp.bfloat16)
        acc = jnp.zeros((M2, D_MODEL), jnp.float32)
        for j in [0, 1, 3, 2]:
            if j == 0:
                wq_g = wq_ref[...]
            else:
                wait_recv(wq_buf, 0, j)
                wq_g = wq_buf[j]

            g = lax.rem(my_pos + j, N_DEV)
            q_all = jnp.dot(x2, wq_g,
                            preferred_element_type=jnp.float32)
            q_all = q_all.astype(jnp.bfloat16)
            ctx_parts = []
            for b in range(B_PER):
                for h in range(HQ_PER):
                    head = g * HQ_PER + h
                    q = q_all[b * SQ:(b + 1) * SQ, h * DH:(h + 1) * DH]
                    kk = k_ref[head, b]
                    s = lax.dot_general(
                        q, kk, (((1,), (1,)), ((), ())),
                        preferred_element_type=jnp.float32,
                    ).astype(jnp.bfloat16) * jnp.bfloat16(SCALE)
                    e = jnp.exp(jnp.where(mask, s, jnp.bfloat16(-1e9)))
                    r = 1.0 / jnp.sum(e, axis=1, keepdims=True,
                                      dtype=jnp.float32)
                    ctx_h = jnp.dot(e, v_ref[head, b],
                                    preferred_element_type=jnp.float32)
                    ctx_parts.append(ctx_h * r)
            ctx = jnp.concatenate(
                [jnp.concatenate(ctx_parts[b * HQ_PER:(b + 1) * HQ_PER], axis=1)
                 for b in range(B_PER)], axis=0)

            if j == 0:
                wo_g = wo_ref[...]
            else:
                wait_recv(wo_buf, 1, j)
                wo_g = wo_buf[j]
            acc = acc + jnp.dot(ctx.astype(jnp.bfloat16), wo_g,
                                preferred_element_type=jnp.float32)

        out_ref[...] = acc.reshape(B_PER, SQ, D_MODEL)

        for i in range(N_DEV - 1):
            dq = pltpu.make_async_remote_copy(
                src_ref=wq_ref, dst_ref=wq_buf.at[0],
                send_sem=send_sems.at[0, i],
                recv_sem=recv_sems.at[0, 0],
                device_id=(my_pos,), device_id_type=pl.DeviceIdType.MESH,
            )
            dq.wait_send()
            do = pltpu.make_async_remote_copy(
                src_ref=wo_ref, dst_ref=wo_buf.at[0],
                send_sem=send_sems.at[1, i],
                recv_sem=recv_sems.at[1, 0],
                device_id=(my_pos,), device_id_type=pl.DeviceIdType.MESH,
            )
            do.wait_send()

    return pl.pallas_call(
        body,
        out_shape=jax.ShapeDtypeStruct((B_PER, SQ, D_MODEL), jnp.float32),
        in_specs=[
            pl.BlockSpec(memory_space=pltpu.VMEM),
            pl.BlockSpec(memory_space=pltpu.VMEM),
            pl.BlockSpec(memory_space=pltpu.VMEM),
            pl.BlockSpec(memory_space=pltpu.VMEM),
            pl.BlockSpec(memory_space=pltpu.VMEM),
        ],
        out_specs=pl.BlockSpec(memory_space=pltpu.VMEM),
        scratch_shapes=[
            pltpu.VMEM((N_DEV, D_MODEL, D_HEADS), jnp.bfloat16),
            pltpu.VMEM((N_DEV, D_HEADS, D_MODEL), jnp.bfloat16),
            pltpu.SemaphoreType.DMA((2, N_DEV - 1)),
            pltpu.SemaphoreType.DMA((2, N_DEV)),
        ],
        compiler_params=pltpu.CompilerParams(collective_id=0),
    )(x, wq_b, k_loc, v_loc, wo_b)
